# baseline (speedup 1.0000x reference)
import functools
import numpy as np
import jax
import jax.numpy as jnp
from jax import lax
from jax.experimental import pallas as pl
from jax.experimental.pallas import tpu as pltpu
from jax.experimental.pallas import tpu_sc as plsc

_HIDDEN = 4096
_NE = 64
_NG = 8
_GS = _NE // _NG
_TKG = 4
_SCALE = 2.5
_NT = 16384

_BT = 1024

_NC, _NS = 2, 16
_NW = _NC * _NS
_TPW = _NT // _NW
_CH = 128

_STEPS = []
for _m in (2, 4, 8, 16, 32, 64):
    _d = _m // 2
    while _d >= 1:
        _STEPS.append((_m, _d))
        _d //= 2


def _monotone_i32(x):
    u = lax.bitcast_convert_type(x, jnp.int32)
    return u ^ (lax.shift_right_arithmetic(u, 31) & jnp.int32(0x7FFFFFFF))


def _tc_body(hs_ref, w_ref, b_ref, key_ref, sc_ref):
    hs = hs_ref[...]
    w = w_ref[...]
    logits = lax.dot_general(w, hs, (((1,), (1,)), ((), ())),
                             preferred_element_type=jnp.float32)
    scores = jax.nn.sigmoid(logits)
    s4c = scores + b_ref[...]

    s3 = s4c.reshape(_NG, _GS, _BT)
    m1 = jnp.max(s3, axis=1)
    i8 = lax.broadcasted_iota(jnp.int32, (_NG, _GS, _BT), 1)
    ismax = s3 == m1[:, None, :]
    firstpos = jnp.min(jnp.where(ismax, i8, _GS), axis=1)
    m2 = jnp.max(jnp.where(i8 == firstpos[:, None, :], -jnp.inf, s3), axis=1)
    gkey = _monotone_i32(m1 + m2)

    tgl = gkey[None, :, :] - gkey[:, None, :]
    ig = lax.broadcasted_iota(jnp.int32, (_NG, _NG, 1), 0)
    ih = lax.broadcasted_iota(jnp.int32, (_NG, _NG, 1), 1)
    mlt = (ih < ig).astype(jnp.int32)
    gcnt = (tgl + mlt) > 0
    grank = jnp.sum(gcnt.astype(jnp.int32), axis=1)
    gmf = (grank < _TKG).astype(jnp.float32)
    smf = jnp.broadcast_to(gmf[:, None, :], (_NG, _GS, _BT)).reshape(_NE, _BT)
    sp = jnp.where(smf > 0, s4c, 0.0)

    key_ref[...] = _monotone_i32(sp)
    sc_ref[...] = scores


def _tc_stage(hs, w, b):
    return pl.pallas_call(
        _tc_body,
        grid=(_NT // _BT,),
        in_specs=[
            pl.BlockSpec((_BT, _HIDDEN), lambda i: (i, 0)),
            pl.BlockSpec((_NE, _HIDDEN), lambda i: (0, 0)),
            pl.BlockSpec((_NE, 1), lambda i: (0, 0)),
        ],
        out_specs=[
            pl.BlockSpec((_NE, _BT), lambda i: (0, i)),
            pl.BlockSpec((_NE, _BT), lambda i: (0, i)),
        ],
        out_shape=[
            jax.ShapeDtypeStruct((_NE, _NT), jnp.int32),
            jax.ShapeDtypeStruct((_NE, _NT), jnp.float32),
        ],
        compiler_params=pltpu.CompilerParams(
            dimension_semantics=("parallel",),
        ),
    )(hs, w, b)


def _sc_route(key_hbm, sc_hbm, idx_hbm, wt_hbm, keyv, scv, idxv, wtv):
    wid = lax.axis_index("s") * _NC + lax.axis_index("c")
    base = wid * _TPW
    for c in range(_TPW // _CH):
        off = base + c * _CH
        pltpu.sync_copy(key_hbm.at[:, pl.ds(off, _CH)], keyv)
        pltpu.sync_copy(sc_hbm.at[:, pl.ds(off, _CH)], scv)

        def init_row(e, _):
            for gg in range(_CH // 16):
                idxv[e, pl.ds(gg * 16, 16)] = jnp.zeros((16,), jnp.int32) + e
            return 0

        lax.fori_loop(0, _NE, init_row, 0)

        def group_body(g, _):
            sl = pl.ds(g * 16, 16)
            for m, d in _STEPS:
                k = d.bit_length() - 1

                def pair_body(p, _, m=m, d=d, k=k):
                    e = ((p >> k) << (k + 1)) | (p & (d - 1))
                    f = e | d
                    ka = keyv[e, sl]
                    kb = keyv[f, sl]
                    ia = idxv[e, sl]
                    ib = idxv[f, sl]
                    sa = scv[e, sl]
                    sb = scv[f, sl]
                    tl = jnp.where(ia < ib, jnp.int32(1), jnp.int32(-1))
                    t = (ka - kb) * 2 + tl
                    dirs = jnp.where((e & m) == 0, jnp.int32(1),
                                     jnp.int32(-1))
                    take_a = (t * dirs) > 0
                    keyv[e, sl] = jnp.where(take_a, ka, kb)
                    keyv[f, sl] = jnp.where(take_a, kb, ka)
                    idxv[e, sl] = jnp.where(take_a, ia, ib)
                    idxv[f, sl] = jnp.where(take_a, ib, ia)
                    scv[e, sl] = jnp.where(take_a, sa, sb)
                    scv[f, sl] = jnp.where(take_a, sb, sa)
                    return 0

                lax.fori_loop(0, _NE // 2, pair_body, 0)

            def sum_body(e, acc):
                return acc + scv[e, sl]

            denom = lax.fori_loop(0, _NE, sum_body,
                                  jnp.zeros((16,), jnp.float32))
            inv = _SCALE / (denom + 1e-20)

            def scale_body(e, _):
                wtv[e, sl] = scv[e, sl] * inv
                return 0

            lax.fori_loop(0, _NE, scale_body, 0)
            return 0

        lax.fori_loop(0, _CH // 16, group_body, 0)

        pltpu.sync_copy(idxv, idx_hbm.at[:, pl.ds(off, _CH)])
        pltpu.sync_copy(wtv, wt_hbm.at[:, pl.ds(off, _CH)])


_sc_route_call = functools.partial(
    pl.kernel,
    mesh=plsc.VectorSubcoreMesh(core_axis_name="c", subcore_axis_name="s"),
    out_type=[
        jax.ShapeDtypeStruct((_NE, _NT), jnp.int32),
        jax.ShapeDtypeStruct((_NE, _NT), jnp.float32),
    ],
    scratch_types=[
        pltpu.VMEM((_NE, _CH), jnp.int32),
        pltpu.VMEM((_NE, _CH), jnp.float32),
        pltpu.VMEM((_NE, _CH), jnp.int32),
        pltpu.VMEM((_NE, _CH), jnp.float32),
    ],
)(_sc_route)


def kernel(hidden_states, weight, e_score_correction_bias):
    hs = hidden_states.reshape(-1, _HIDDEN).astype(jnp.float32)
    w = weight.astype(jnp.float32)
    b = e_score_correction_bias.astype(jnp.float32).reshape(_NE, 1)
    key_t, sc_t = _tc_stage(hs, w, b)
    idx_t, wt_t = _sc_route_call(key_t, sc_t)
    return (idx_t.T, wt_t.T)


# speedup vs baseline: 3.0708x; 3.0708x over previous
import numpy as np
import jax
import jax.numpy as jnp
from jax import lax
from jax.experimental import pallas as pl
from jax.experimental.pallas import tpu as pltpu

_HIDDEN = 4096
_NE = 64
_NG = 8
_GS = _NE // _NG
_TKG = 4
_SCALE = 2.5
_NT = 16384

_BT = 1024

_STEPS = []
for _m in (2, 4, 8, 16, 32, 64):
    _d = _m // 2
    while _d >= 1:
        _STEPS.append((_m, _d))
        _d //= 2


def _monotone_i32(x):
    u = lax.bitcast_convert_type(x, jnp.int32)
    return u ^ (lax.shift_right_arithmetic(u, 31) & jnp.int32(0x7FFFFFFF))


def _swap_ax0(x, d):
    r = x.reshape(8 // (2 * d), 2, d, 8, x.shape[-1])
    return jnp.concatenate((r[:, 1:2], r[:, 0:1]), axis=1).reshape(x.shape)


def _swap_ax1(x, d):
    r = x.reshape(8, 8 // (2 * d), 2, d, x.shape[-1])
    return jnp.concatenate((r[:, :, 1:2], r[:, :, 0:1]), axis=2).reshape(x.shape)


def _router_body(hs_ref, w_ref, b_ref, idx_ref, wt_ref):
    hs = hs_ref[...]
    w = w_ref[...]
    logits = lax.dot_general(w, hs, (((1,), (1,)), ((), ())),
                             preferred_element_type=jnp.float32)
    scores = jax.nn.sigmoid(logits)
    s4c = scores + b_ref[...]

    s3 = s4c.reshape(_NG, _GS, _BT)
    m1 = jnp.max(s3, axis=1)
    i8 = lax.broadcasted_iota(jnp.int32, (_NG, _GS, _BT), 1)
    ismax = s3 == m1[:, None, :]
    firstpos = jnp.min(jnp.where(ismax, i8, _GS), axis=1)
    m2 = jnp.max(jnp.where(i8 == firstpos[:, None, :], -jnp.inf, s3), axis=1)
    gkey = _monotone_i32(m1 + m2)

    tgl = gkey[None, :, :] - gkey[:, None, :]
    ig = lax.broadcasted_iota(jnp.int32, (_NG, _NG, 1), 0)
    ih = lax.broadcasted_iota(jnp.int32, (_NG, _NG, 1), 1)
    mlt = (ih < ig).astype(jnp.int32)
    gcnt = (tgl + mlt) > 0
    grank = jnp.sum(gcnt.astype(jnp.int32), axis=1)
    gmf = (grank < _TKG).astype(jnp.float32)
    smf = jnp.broadcast_to(gmf[:, None, :], (_NG, _GS, _BT)).reshape(_NE, _BT)
    sp = jnp.where(smf > 0, s4c, 0.0)

    key = jnp.swapaxes(_monotone_i32(sp).reshape(_NG, _GS, _BT), 0, 1)
    sc = jnp.swapaxes(scores.reshape(_NG, _GS, _BT), 0, 1)
    i0 = lax.broadcasted_iota(jnp.int32, (8, 8, _BT), 0)
    i1 = lax.broadcasted_iota(jnp.int32, (8, 8, _BT), 1)
    idx = 8 * i1 + i0
    ii0 = lax.broadcasted_iota(jnp.int32, (8, 8, 1), 0)
    ii1 = lax.broadcasted_iota(jnp.int32, (8, 8, 1), 1)
    inet = 8 * ii1 + ii0
    for m, d in _STEPS:
        kf = ((inet & d) == 0) == ((inet & m) == 0)
        dirsign = jnp.where(kf, jnp.int32(1), jnp.int32(-1))
        swap = _swap_ax0 if d < 8 else _swap_ax1
        dd = d if d < 8 else d // 8
        kp = swap(key, dd)
        ip = swap(idx, dd)
        scp = swap(sc, dd)
        tl = jnp.where(idx < ip, jnp.int32(1), jnp.int32(-1))
        t_adj = (key - kp) * 2 + tl
        take_self = (t_adj * dirsign) > 0
        key = jnp.where(take_self, key, kp)
        idx = jnp.where(take_self, idx, ip)
        sc = jnp.where(take_self, sc, scp)
    idx_s = jnp.swapaxes(idx, 0, 1).reshape(_NE, _BT)
    sc_s = jnp.swapaxes(sc, 0, 1).reshape(_NE, _BT)

    denom = jnp.sum(sc_s, axis=0, keepdims=True) + 1e-20
    wt = (sc_s / denom) * _SCALE

    idx_ref[...] = idx_s
    wt_ref[...] = wt


def kernel(hidden_states, weight, e_score_correction_bias):
    hs = hidden_states.reshape(-1, _HIDDEN).astype(jnp.float32)
    w = weight.astype(jnp.float32)
    b = e_score_correction_bias.astype(jnp.float32).reshape(_NE, 1)
    grid = (_NT // _BT,)
    idx_t, wt_t = pl.pallas_call(
        _router_body,
        grid=grid,
        in_specs=[
            pl.BlockSpec((_BT, _HIDDEN), lambda i: (i, 0)),
            pl.BlockSpec((_NE, _HIDDEN), lambda i: (0, 0)),
            pl.BlockSpec((_NE, 1), lambda i: (0, 0)),
        ],
        out_specs=[
            pl.BlockSpec((_NE, _BT), lambda i: (0, i)),
            pl.BlockSpec((_NE, _BT), lambda i: (0, i)),
        ],
        out_shape=[
            jax.ShapeDtypeStruct((_NE, _NT), jnp.int32),
            jax.ShapeDtypeStruct((_NE, _NT), jnp.float32),
        ],
        compiler_params=pltpu.CompilerParams(
            dimension_semantics=("parallel",),
        ),
    )(hs, w, b)
    return (idx_t.T, wt_t.T)
